# baseline (device time: 86137 ns/iter reference)
import functools

import jax
import jax.numpy as jnp
from jax import lax
from jax.experimental import pallas as pl
from jax.experimental.pallas import tpu as pltpu

N_DEV = 4


def kernel(A, B):
    m_per, k = A.shape
    k2, n = B.shape
    assert k == k2

    def body(a_ref, b_ref, out_ref, comm_ref, send_sems, recv_sems):
        my_pos = lax.axis_index("i")
        left = (my_pos - 1) % N_DEV
        right = (my_pos + 1) % N_DEV

        barrier_sem = pltpu.get_barrier_semaphore()
        for nbr in [left, right]:
            pl.semaphore_signal(
                barrier_sem, inc=1,
                device_id=(nbr,), device_id_type=pl.DeviceIdType.MESH,
            )
        pl.semaphore_wait(barrier_sem, 2)

        rdmas = []
        for h in range(N_DEV - 1):
            src = a_ref if h == 0 else comm_ref.at[h - 1]
            rdma = pltpu.make_async_remote_copy(
                src_ref=src,
                dst_ref=comm_ref.at[h],
                send_sem=send_sems.at[h],
                recv_sem=recv_sems.at[h],
                device_id=(right,),
                device_id_type=pl.DeviceIdType.MESH,
            )
            rdma.start()
            rdmas.append(rdma)

            origin = (my_pos - h) % N_DEV
            chunk = a_ref[:, :] if h == 0 else comm_ref[h - 1, :, :]
            out_ref[pl.ds(origin * m_per, m_per), :] = jnp.dot(
                chunk, b_ref[:, :], preferred_element_type=jnp.float32
            )
            rdma.wait()

        origin = (my_pos - (N_DEV - 1)) % N_DEV
        out_ref[pl.ds(origin * m_per, m_per), :] = jnp.dot(
            comm_ref[N_DEV - 2, :, :], b_ref[:, :],
            preferred_element_type=jnp.float32,
        )

        @functools.partial(
            pl.run_scoped, second_barrier=pltpu.SemaphoreType.REGULAR
        )
        def _(second_barrier):
            for nbr in [left, right]:
                pl.semaphore_signal(
                    second_barrier, inc=1,
                    device_id=(nbr,), device_id_type=pl.DeviceIdType.MESH,
                )
            pl.semaphore_wait(second_barrier, 2)

    return pl.pallas_call(
        body,
        out_shape=jax.ShapeDtypeStruct((N_DEV * m_per, n), jnp.float32),
        in_specs=[
            pl.BlockSpec(memory_space=pltpu.VMEM),
            pl.BlockSpec(memory_space=pltpu.VMEM),
        ],
        out_specs=pl.BlockSpec(memory_space=pltpu.VMEM),
        scratch_shapes=[
            pltpu.VMEM((N_DEV - 1, m_per, k), jnp.float32),
            pltpu.SemaphoreType.DMA((N_DEV - 1,)),
            pltpu.SemaphoreType.DMA((N_DEV - 1,)),
        ],
        compiler_params=pltpu.CompilerParams(collective_id=0),
    )(A, B)


# device time: 35352 ns/iter; 2.4366x vs baseline; 2.4366x over previous
import functools

import jax
import jax.numpy as jnp
from jax import lax
from jax.experimental import pallas as pl
from jax.experimental.pallas import tpu as pltpu

N_DEV = 4


def kernel(A, B):
    m_per, k = A.shape
    k2, n = B.shape
    assert k == k2
    m_half = m_per // 2

    def body(
        a_ref, b_ref, out_ref,
        my_bf, b_bf, recv_l0, recv_r0, recv_l1, recv_r1,
        send_sems, recv_sems,
    ):
        my_pos = lax.axis_index("i")
        left = (my_pos - 1) % N_DEV
        right = (my_pos + 1) % N_DEV

        barrier_sem = pltpu.get_barrier_semaphore()
        for nbr in [left, right]:
            pl.semaphore_signal(
                barrier_sem, inc=1,
                device_id=(nbr,), device_id_type=pl.DeviceIdType.MESH,
            )
        pl.semaphore_wait(barrier_sem, 2)

        my_bf[:, :] = a_ref[:, :].astype(jnp.bfloat16)
        b_bf[:, :] = b_ref[:, :].astype(jnp.bfloat16)

        p1_r = pltpu.make_async_remote_copy(
            src_ref=my_bf, dst_ref=recv_l0,
            send_sem=send_sems.at[0], recv_sem=recv_sems.at[0],
            device_id=(right,), device_id_type=pl.DeviceIdType.MESH,
        )
        p1_l = pltpu.make_async_remote_copy(
            src_ref=my_bf, dst_ref=recv_r0,
            send_sem=send_sems.at[1], recv_sem=recv_sems.at[1],
            device_id=(left,), device_id_type=pl.DeviceIdType.MESH,
        )
        p1_r.start()
        p1_l.start()

        out_ref[pl.ds(my_pos * m_per, m_per), :] = jnp.dot(
            my_bf[:, :], b_bf[:, :], preferred_element_type=jnp.float32
        )

        p1_r.wait_recv()
        p2_r = pltpu.make_async_remote_copy(
            src_ref=recv_l0.at[pl.ds(0, m_half), :],
            dst_ref=recv_l1,
            send_sem=send_sems.at[2], recv_sem=recv_sems.at[2],
            device_id=(right,), device_id_type=pl.DeviceIdType.MESH,
        )
        p2_r.start()
        out_ref[pl.ds(left * m_per, m_per), :] = jnp.dot(
            recv_l0[:, :], b_bf[:, :], preferred_element_type=jnp.float32
        )

        p1_l.wait_recv()
        p2_l = pltpu.make_async_remote_copy(
            src_ref=recv_r0.at[pl.ds(m_half, m_half), :],
            dst_ref=recv_r1,
            send_sem=send_sems.at[3], recv_sem=recv_sems.at[3],
            device_id=(left,), device_id_type=pl.DeviceIdType.MESH,
        )
        p2_l.start()
        out_ref[pl.ds(right * m_per, m_per), :] = jnp.dot(
            recv_r0[:, :], b_bf[:, :], preferred_element_type=jnp.float32
        )

        diag = (my_pos + 2) % N_DEV
        p2_r.wait_recv()
        out_ref[pl.ds(diag * m_per, m_half), :] = jnp.dot(
            recv_l1[:, :], b_bf[:, :], preferred_element_type=jnp.float32
        )
        p2_l.wait_recv()
        out_ref[pl.ds(diag * m_per + m_half, m_half), :] = jnp.dot(
            recv_r1[:, :], b_bf[:, :], preferred_element_type=jnp.float32
        )

        p1_r.wait_send()
        p1_l.wait_send()
        p2_r.wait_send()
        p2_l.wait_send()

        @functools.partial(
            pl.run_scoped, second_barrier=pltpu.SemaphoreType.REGULAR
        )
        def _(second_barrier):
            for nbr in [left, right]:
                pl.semaphore_signal(
                    second_barrier, inc=1,
                    device_id=(nbr,), device_id_type=pl.DeviceIdType.MESH,
                )
            pl.semaphore_wait(second_barrier, 2)

    return pl.pallas_call(
        body,
        out_shape=jax.ShapeDtypeStruct((N_DEV * m_per, n), jnp.float32),
        in_specs=[
            pl.BlockSpec(memory_space=pltpu.VMEM),
            pl.BlockSpec(memory_space=pltpu.VMEM),
        ],
        out_specs=pl.BlockSpec(memory_space=pltpu.VMEM),
        scratch_shapes=[
            pltpu.VMEM((m_per, k), jnp.bfloat16),
            pltpu.VMEM((k, n), jnp.bfloat16),
            pltpu.VMEM((m_per, k), jnp.bfloat16),
            pltpu.VMEM((m_per, k), jnp.bfloat16),
            pltpu.VMEM((m_half, k), jnp.bfloat16),
            pltpu.VMEM((m_half, k), jnp.bfloat16),
            pltpu.SemaphoreType.DMA((4,)),
            pltpu.SemaphoreType.DMA((4,)),
        ],
        compiler_params=pltpu.CompilerParams(collective_id=0),
    )(A, B)


# device time: 35036 ns/iter; 2.4585x vs baseline; 1.0090x over previous
import functools

import jax
import jax.numpy as jnp
from jax import lax
from jax.experimental import pallas as pl
from jax.experimental.pallas import tpu as pltpu

N_DEV = 4


def kernel(A, B):
    m_per, k = A.shape
    k2, n = B.shape
    assert k == k2
    m_half = m_per // 2

    def body(
        a_ref, b_ref, out_ref,
        my_bf, b_bf, recv_l0, recv_r0, recv_l1, recv_r1, out_vmem,
        send_sems, recv_sems, copy_sems,
    ):
        my_pos = lax.axis_index("i")
        left = (my_pos - 1) % N_DEV
        right = (my_pos + 1) % N_DEV

        barrier_sem = pltpu.get_barrier_semaphore()
        for nbr in [left, right]:
            pl.semaphore_signal(
                barrier_sem, inc=1,
                device_id=(nbr,), device_id_type=pl.DeviceIdType.MESH,
            )
        pl.semaphore_wait(barrier_sem, 2)

        my_bf[:, :] = a_ref[:, :].astype(jnp.bfloat16)
        b_bf[:, :] = b_ref[:, :].astype(jnp.bfloat16)

        p1_r = pltpu.make_async_remote_copy(
            src_ref=my_bf, dst_ref=recv_l0,
            send_sem=send_sems.at[0], recv_sem=recv_sems.at[0],
            device_id=(right,), device_id_type=pl.DeviceIdType.MESH,
        )
        p1_l = pltpu.make_async_remote_copy(
            src_ref=my_bf, dst_ref=recv_r0,
            send_sem=send_sems.at[1], recv_sem=recv_sems.at[1],
            device_id=(left,), device_id_type=pl.DeviceIdType.MESH,
        )
        p1_r.start()
        p1_l.start()

        def store_block(slot, origin, rows=m_per):
            copy = pltpu.make_async_copy(
                out_vmem.at[slot, pl.ds(0, rows), :],
                out_ref.at[pl.ds(origin * m_per, rows), :],
                copy_sems.at[slot],
            )
            copy.start()
            return copy

        out_vmem[0, :, :] = jnp.dot(
            my_bf[:, :], b_bf[:, :], preferred_element_type=jnp.float32
        )
        c0 = store_block(0, my_pos)

        p1_r.wait_recv()
        p2_r = pltpu.make_async_remote_copy(
            src_ref=recv_l0.at[pl.ds(0, m_half), :],
            dst_ref=recv_l1,
            send_sem=send_sems.at[2], recv_sem=recv_sems.at[2],
            device_id=(right,), device_id_type=pl.DeviceIdType.MESH,
        )
        p2_r.start()
        p1_l.wait_recv()
        p2_l = pltpu.make_async_remote_copy(
            src_ref=recv_r0.at[pl.ds(m_half, m_half), :],
            dst_ref=recv_r1,
            send_sem=send_sems.at[3], recv_sem=recv_sems.at[3],
            device_id=(left,), device_id_type=pl.DeviceIdType.MESH,
        )
        p2_l.start()

        out_vmem[1, :, :] = jnp.dot(
            recv_l0[:, :], b_bf[:, :], preferred_element_type=jnp.float32
        )
        c1 = store_block(1, left)
        out_vmem[2, :, :] = jnp.dot(
            recv_r0[:, :], b_bf[:, :], preferred_element_type=jnp.float32
        )
        c2 = store_block(2, right)

        diag = (my_pos + 2) % N_DEV
        p2_r.wait_recv()
        out_vmem[3, pl.ds(0, m_half), :] = jnp.dot(
            recv_l1[:, :], b_bf[:, :], preferred_element_type=jnp.float32
        )
        p2_l.wait_recv()
        out_vmem[3, pl.ds(m_half, m_half), :] = jnp.dot(
            recv_r1[:, :], b_bf[:, :], preferred_element_type=jnp.float32
        )
        c3 = store_block(3, diag)

        c0.wait()
        c1.wait()
        c2.wait()
        c3.wait()
        p1_r.wait_send()
        p1_l.wait_send()
        p2_r.wait_send()
        p2_l.wait_send()

        @functools.partial(
            pl.run_scoped, second_barrier=pltpu.SemaphoreType.REGULAR
        )
        def _(second_barrier):
            for nbr in [left, right]:
                pl.semaphore_signal(
                    second_barrier, inc=1,
                    device_id=(nbr,), device_id_type=pl.DeviceIdType.MESH,
                )
            pl.semaphore_wait(second_barrier, 2)

    return pl.pallas_call(
        body,
        out_shape=jax.ShapeDtypeStruct((N_DEV * m_per, n), jnp.float32),
        in_specs=[
            pl.BlockSpec(memory_space=pltpu.VMEM),
            pl.BlockSpec(memory_space=pltpu.VMEM),
        ],
        out_specs=pl.BlockSpec(memory_space=pl.ANY),
        scratch_shapes=[
            pltpu.VMEM((m_per, k), jnp.bfloat16),
            pltpu.VMEM((k, n), jnp.bfloat16),
            pltpu.VMEM((m_per, k), jnp.bfloat16),
            pltpu.VMEM((m_per, k), jnp.bfloat16),
            pltpu.VMEM((m_half, k), jnp.bfloat16),
            pltpu.VMEM((m_half, k), jnp.bfloat16),
            pltpu.VMEM((N_DEV, m_per, n), jnp.float32),
            pltpu.SemaphoreType.DMA((4,)),
            pltpu.SemaphoreType.DMA((4,)),
            pltpu.SemaphoreType.DMA((4,)),
        ],
        compiler_params=pltpu.CompilerParams(collective_id=0),
    )(A, B)


# device time: 31423 ns/iter; 2.7412x vs baseline; 1.1150x over previous
import jax
import jax.numpy as jnp
from jax import lax
from jax.experimental import pallas as pl
from jax.experimental.pallas import tpu as pltpu

N_DEV = 4


def kernel(A, B):
    m_per, k = A.shape
    k2, n = B.shape
    assert k == k2
    m_half = m_per // 2
    TOP = pl.ds(0, m_half)
    BOT = pl.ds(m_half, m_half)

    def body(
        a_ref, b_ref, out_ref,
        my_bf, b_bf, recv_l, recv_r, recv_d, out_vmem,
        send_sems, recv_sems, copy_sems,
    ):
        my_pos = lax.axis_index("i")
        left = (my_pos - 1) % N_DEV
        right = (my_pos + 1) % N_DEV

        barrier_sem = pltpu.get_barrier_semaphore()
        for nbr in [left, right]:
            pl.semaphore_signal(
                barrier_sem, inc=1,
                device_id=(nbr,), device_id_type=pl.DeviceIdType.MESH,
            )
        my_bf[:, :] = a_ref[:, :].astype(jnp.bfloat16)
        pl.semaphore_wait(barrier_sem, 2)

        def rdma(i, src, dst, dev):
            return pltpu.make_async_remote_copy(
                src_ref=src, dst_ref=dst,
                send_sem=send_sems.at[i], recv_sem=recv_sems.at[i],
                device_id=(dev,), device_id_type=pl.DeviceIdType.MESH,
            )

        s_rt = rdma(0, my_bf.at[TOP, :], recv_l.at[TOP, :], right)
        s_lb = rdma(1, my_bf.at[BOT, :], recv_r.at[BOT, :], left)
        s_rb = rdma(2, my_bf.at[BOT, :], recv_l.at[BOT, :], right)
        s_lt = rdma(3, my_bf.at[TOP, :], recv_r.at[TOP, :], left)
        s_rt.start()
        s_lb.start()
        s_rb.start()
        s_lt.start()

        b_bf[:, :] = b_ref[:, :].astype(jnp.bfloat16)

        def mm(slot, rows, chunk_rows):
            out_vmem[slot, rows, :] = jnp.dot(
                chunk_rows, b_bf[:, :], preferred_element_type=jnp.float32
            )

        def store_half(sem_i, slot, rows, origin, row_off):
            copy = pltpu.make_async_copy(
                out_vmem.at[slot, rows, :],
                out_ref.at[pl.ds(origin * m_per + row_off, m_half), :],
                copy_sems.at[sem_i],
            )
            copy.start()
            return copy

        mm(0, TOP, my_bf[TOP, :])
        c0 = store_half(0, 0, TOP, my_pos, 0)
        mm(0, BOT, my_bf[BOT, :])
        c1 = store_half(1, 0, BOT, my_pos, m_half)

        s_rt.wait_recv()
        f_r = rdma(4, recv_l.at[TOP, :], recv_d.at[TOP, :], right)
        f_r.start()
        mm(1, TOP, recv_l[TOP, :])
        c2 = store_half(2, 1, TOP, left, 0)

        s_lb.wait_recv()
        f_l = rdma(5, recv_r.at[BOT, :], recv_d.at[BOT, :], left)
        f_l.start()
        mm(2, BOT, recv_r[BOT, :])
        c3 = store_half(3, 2, BOT, right, m_half)

        s_rb.wait_recv()
        mm(1, BOT, recv_l[BOT, :])
        c4 = store_half(4, 1, BOT, left, m_half)
        s_lt.wait_recv()
        mm(2, TOP, recv_r[TOP, :])
        c5 = store_half(5, 2, TOP, right, 0)

        diag = (my_pos + 2) % N_DEV
        f_r.wait_recv()
        mm(3, TOP, recv_d[TOP, :])
        c6 = store_half(6, 3, TOP, diag, 0)
        f_l.wait_recv()
        mm(3, BOT, recv_d[BOT, :])
        c7 = store_half(7, 3, BOT, diag, m_half)

        for c in [c0, c1, c2, c3, c4, c5, c6, c7]:
            c.wait()
        for s in [s_rt, s_lb, s_rb, s_lt, f_r, f_l]:
            s.wait_send()

    return pl.pallas_call(
        body,
        out_shape=jax.ShapeDtypeStruct((N_DEV * m_per, n), jnp.float32),
        in_specs=[
            pl.BlockSpec(memory_space=pltpu.VMEM),
            pl.BlockSpec(memory_space=pltpu.VMEM),
        ],
        out_specs=pl.BlockSpec(memory_space=pl.ANY),
        scratch_shapes=[
            pltpu.VMEM((m_per, k), jnp.bfloat16),
            pltpu.VMEM((k, n), jnp.bfloat16),
            pltpu.VMEM((m_per, k), jnp.bfloat16),
            pltpu.VMEM((m_per, k), jnp.bfloat16),
            pltpu.VMEM((m_per, k), jnp.bfloat16),
            pltpu.VMEM((N_DEV, m_per, n), jnp.float32),
            pltpu.SemaphoreType.DMA((6,)),
            pltpu.SemaphoreType.DMA((6,)),
            pltpu.SemaphoreType.DMA((8,)),
        ],
        compiler_params=pltpu.CompilerParams(collective_id=0),
    )(A, B)
